# baseline (speedup 1.0000x reference)
import functools

import jax
import jax.numpy as jnp
from jax import lax
from jax.experimental import pallas as pl
from jax.experimental.pallas import tpu as pltpu
from jax.experimental.pallas import tpu_sc as plsc

X_RANGE = (-50.0, 50.0)
Y_RANGE = (-50.0, 50.0)
Z_RANGE = (-3.0, 5.0)
BEV_SIZE = 256
NUM_FEATURES = 64
X_SIZE = (X_RANGE[1] - X_RANGE[0]) / BEV_SIZE
Y_SIZE = (Y_RANGE[1] - Y_RANGE[0]) / BEV_SIZE

B = 8
N = 100000
NBINS = BEV_SIZE * BEV_SIZE
NRANGES = 4
RBINS = NBINS // NRANGES
CHUNK = 5000
NCHUNKS = N // CHUNK
NPAIRS = NCHUNKS // 2
L = 16
UNROLL = 5
ZBUF = 16384
ZSPAN = 60 * NBINS // NRANGES
NZ = ZSPAN // ZBUF
ZPER = NZ // NPAIRS


def _prep(pxf, pyf, pzf):
    LB = 12800

    def body(px_ref, py_ref, pz_ref, o_ref):
        x = px_ref[...]
        y = py_ref[...]
        z = pz_ref[...]
        m = (x >= X_RANGE[0]) & (x < X_RANGE[1]) & \
            (y >= Y_RANGE[0]) & (y < Y_RANGE[1]) & \
            (z >= Z_RANGE[0]) & (z < Z_RANGE[1])
        xi = jnp.clip(((x - X_RANGE[0]) / X_SIZE).astype(jnp.int32), 0, BEV_SIZE - 1)
        yi = jnp.clip(((y - Y_RANGE[0]) / Y_SIZE).astype(jnp.int32), 0, BEV_SIZE - 1)
        o_ref[...] = jnp.where(m, yi * BEV_SIZE + xi, -1)

    return pl.pallas_call(
        body,
        grid=(pl.cdiv(N, LB),),
        in_specs=[pl.BlockSpec((B, LB), lambda i: (0, i))] * 3,
        out_specs=pl.BlockSpec((B, LB), lambda i: (0, i)),
        out_shape=jax.ShapeDtypeStruct((B, N), jnp.int32),
    )(pxf, pyf, pzf)


def _sc_scatter(idx_flat, z_flat, f_flat):
    mesh = plsc.VectorSubcoreMesh(core_axis_name="c", subcore_axis_name="s")

    @functools.partial(
        pl.kernel,
        mesh=mesh,
        out_type=(
            jax.ShapeDtypeStruct((B * 3 * NBINS,), jnp.float32),
            jax.ShapeDtypeStruct((B * NUM_FEATURES * NBINS,), jnp.float32),
        ),
        compiler_params=pltpu.CompilerParams(
            needs_layout_passes=False,
            use_tc_tiling_on_sc=False,
        ),
        scratch_types=[
            pltpu.VMEM((CHUNK,), jnp.int32),
            pltpu.VMEM((CHUNK,), jnp.float32),
            pltpu.VMEM((CHUNK,), jnp.float32),
            pltpu.VMEM((CHUNK,), jnp.int32),
            pltpu.VMEM((CHUNK,), jnp.float32),
            pltpu.VMEM((CHUNK,), jnp.float32),
            pltpu.VMEM((RBINS,), jnp.float32),
            pltpu.VMEM((RBINS,), jnp.float32),
            pltpu.VMEM((RBINS,), jnp.float32),
            pltpu.VMEM((ZBUF,), jnp.float32),
            pltpu.SemaphoreType.DMA,
            pltpu.SemaphoreType.DMA,
            pltpu.SemaphoreType.DMA,
        ],
    )
    def k(idx_hbm, z_hbm, f_hbm, acc_hbm, canvas_hbm,
          idx_a, z_a, f_a, idx_b, z_b, f_b, acc_c, acc_z, acc_f, zbuf,
          sem_a, sem_b, sem_zw):
        c = lax.axis_index("c")
        s = lax.axis_index("s")
        wid = s * 2 + c
        b = wid // NRANGES
        r = wid % NRANGES
        lo = r * RBINS

        zeros16 = jnp.zeros((L,), jnp.float32)

        @plsc.parallel_loop(0, RBINS, L, unroll=8)
        def _(o):
            acc_c[pl.ds(o, L)] = zeros16
            acc_z[pl.ds(o, L)] = zeros16
            acc_f[pl.ds(o, L)] = zeros16

        @plsc.parallel_loop(0, ZBUF, L, unroll=8)
        def _(o):
            zbuf[pl.ds(o, L)] = zeros16

        ones16 = jnp.ones((L,), jnp.float32)
        base = b * N
        zbase = b * NUM_FEATURES * NBINS + 4 * NBINS + r * ZSPAN

        def fire_zeros(p):
            for t in range(ZPER):
                off = zbase + (p * ZPER + t) * ZBUF
                pltpu.make_async_copy(
                    zbuf, canvas_hbm.at[pl.ds(off, ZBUF)], sem_zw
                ).start()

        def issue(g, bi, bz, bf, sem):
            off = base + g * CHUNK
            pltpu.make_async_copy(idx_hbm.at[pl.ds(off, CHUNK)], bi, sem).start()
            pltpu.make_async_copy(z_hbm.at[pl.ds(off, CHUNK)], bz, sem).start()
            pltpu.make_async_copy(f_hbm.at[pl.ds(off, CHUNK)], bf, sem).start()

        def wait(bi, bz, bf, sem):
            pltpu.make_async_copy(idx_hbm.at[pl.ds(base, CHUNK)], bi, sem).wait()
            pltpu.make_async_copy(z_hbm.at[pl.ds(base, CHUNK)], bz, sem).wait()
            pltpu.make_async_copy(f_hbm.at[pl.ds(base, CHUNK)], bf, sem).wait()

        def process(bi, bz, bf):
            @plsc.parallel_loop(0, CHUNK, L, unroll=UNROLL)
            def _(o):
                iv = bi[pl.ds(o, L)]
                li = iv - lo
                m = plsc.bitcast(li, jnp.uint32) < jnp.uint32(RBINS)
                plsc.addupdate_scatter(acc_c, [li], ones16, mask=m)
                plsc.addupdate_scatter(acc_z, [li], bz[pl.ds(o, L)], mask=m)
                plsc.addupdate_scatter(acc_f, [li], bf[pl.ds(o, L)], mask=m)

        issue(0, idx_a, z_a, f_a, sem_a)

        def pair_body(p, carry):
            g = p * 2
            wait(idx_a, z_a, f_a, sem_a)
            issue(g + 1, idx_b, z_b, f_b, sem_b)
            fire_zeros(p)
            process(idx_a, z_a, f_a)

            @pl.when(g + 2 < NCHUNKS)
            def _():
                issue(g + 2, idx_a, z_a, f_a, sem_a)

            wait(idx_b, z_b, f_b, sem_b)
            process(idx_b, z_b, f_b)
            return carry

        lax.fori_loop(0, NPAIRS, pair_body, 0)

        obase = b * 3 * NBINS + lo
        pltpu.sync_copy(acc_c, acc_hbm.at[pl.ds(obase, RBINS)])
        pltpu.sync_copy(acc_z, acc_hbm.at[pl.ds(obase + NBINS, RBINS)])
        pltpu.sync_copy(acc_f, acc_hbm.at[pl.ds(obase + 2 * NBINS, RBINS)])

        pltpu.make_async_copy(
            canvas_hbm.at[pl.ds(zbase, ZSPAN)],
            canvas_hbm.at[pl.ds(zbase, ZSPAN)],
            sem_zw,
        ).wait()

    return k(idx_flat, z_flat, f_flat)


def _finalize(acc, canvas):
    accr = acc.reshape(B, 3, 512, 128)
    canvasr = canvas.reshape(B, NUM_FEATURES, 512, 128)

    def body(acc_ref, _, o_ref):
        cnt = acc_ref[0, 0]
        zs = acc_ref[0, 1]
        fs = acc_ref[0, 2]
        ch0 = jnp.log1p(cnt)
        denom = jnp.maximum(jnp.exp(ch0), 1.0)
        o_ref[0, 0] = ch0
        o_ref[0, 1] = zs
        o_ref[0, 2] = zs / denom
        o_ref[0, 3] = fs / denom

    out = pl.pallas_call(
        body,
        grid=(B,),
        in_specs=[
            pl.BlockSpec((1, 3, 512, 128), lambda b: (b, 0, 0, 0)),
            pl.BlockSpec(memory_space=pl.ANY),
        ],
        out_specs=pl.BlockSpec((1, 4, 512, 128), lambda b: (b, 0, 0, 0)),
        out_shape=jax.ShapeDtypeStruct((B, NUM_FEATURES, 512, 128), jnp.float32),
        input_output_aliases={1: 0},
    )(accr, canvasr)
    return out.reshape(B, NUM_FEATURES, BEV_SIZE, BEV_SIZE)


def kernel(points):
    px = points[:, :, 0]
    py = points[:, :, 1]
    pz = points[:, :, 2]
    pf = points[:, :, 3]
    idx = _prep(px, py, pz)
    acc, canvas = _sc_scatter(idx.reshape(-1), pz.reshape(-1), pf.reshape(-1))
    return _finalize(acc, canvas)
